# baseline (speedup 1.0000x reference)
import functools

import jax
import jax.numpy as jnp
from jax import lax
from jax.experimental import pallas as pl
from jax.experimental.pallas import tpu as pltpu
from jax.experimental.pallas import tpu_sc as plsc

_NUM_CLASSES = 150
_N = 8
_S = 512
_HW = _S * _S
_TOTAL = _N * _HW
_NC, _NS = 2, 16
_E = _TOTAL // (_NC * _NS)
_ROWS = 16
_CH = _ROWS * _S
_NCHUNK = _E // _CH
_L = 16
_BINS = 160
_STRIDE = 161
_HSZ = _L * _STRIDE
_RED = 3 * _BINS

_mesh = plsc.VectorSubcoreMesh(core_axis_name="c", subcore_axis_name="s")


@functools.partial(
    pl.kernel,
    out_type=jax.ShapeDtypeStruct((_NC * _L,), jnp.float32),
    mesh=_mesh,
    compiler_params=pltpu.CompilerParams(
        needs_layout_passes=False, use_tc_tiling_on_sc=True),
    scratch_types=[
        pltpu.VMEM((_ROWS, _S), jnp.int32),
        pltpu.VMEM((_ROWS, _S), jnp.int32),
        pltpu.VMEM((_ROWS, _S), jnp.int32),
        pltpu.VMEM((_ROWS, _S), jnp.int32),
        pltpu.VMEM((_HSZ,), jnp.int32),
        pltpu.VMEM((_HSZ,), jnp.int32),
        pltpu.VMEM((_RED,), jnp.int32),
        pltpu.VMEM_SHARED((_NS * _RED,), jnp.int32),
        pltpu.VMEM((_NS * _RED,), jnp.int32),
        pltpu.VMEM((_L,), jnp.float32),
        pltpu.SemaphoreType.DMA,
        pltpu.SemaphoreType.DMA,
        pltpu.SemaphoreType.DMA,
        pltpu.SemaphoreType.DMA,
    ],
)
def _dice_hist_sc(yp_hbm, yt_hbm, out_hbm, ypb0, ypb1, ytb0, ytb1, hc, ht,
                  red, shared, comb, outv, sp0, sp1, st0, st1):
    c = lax.axis_index("c")
    s = lax.axis_index("s")
    g = c * _NS + s
    n = g >> 2
    row0 = (g & 3) * (_S // 4)

    lane = lax.iota(jnp.int32, _L)
    lane_off = lane * _STRIDE
    ones = jnp.ones((_L,), jnp.int32)
    zeros = jnp.zeros((_L,), jnp.int32)

    bufs = ((ypb0, ytb0, sp0, st0), (ypb1, ytb1, sp1, st1))

    def _issue(k):
        ypb, ytb, sp, st = bufs[k % 2]
        r = row0 + k * _ROWS
        return (
            pltpu.async_copy(yp_hbm.at[n, 0, pl.ds(r, _ROWS), :], ypb, sp),
            pltpu.async_copy(yt_hbm.at[n, 0, pl.ds(r, _ROWS), :], ytb, st))

    pending = _issue(0)

    @plsc.parallel_loop(0, _HSZ // _L, unroll=8)
    def _zero(i):
        hc[pl.ds(i * _L, _L)] = zeros
        ht[pl.ds(i * _L, _L)] = zeros

    for k in range(_NCHUNK):
        ypb, ytb, _, _ = bufs[k % 2]
        hp_, ht_ = pending
        hp_.wait()
        ht_.wait()
        if k + 1 < _NCHUNK:
            pending = _issue(k + 1)

        @plsc.parallel_loop(0, _CH // _L, unroll=8)
        def _body(i):
            r = i >> 5
            col = (i & 31) * _L
            ypv = ypb[r, pl.ds(col, _L)]
            ytv = ytb[r, pl.ds(col, _L)]
            val = jnp.where(ypv == ytv, jnp.int32(0x10001), jnp.int32(1))
            plsc.addupdate_scatter(hc, [ypv + lane_off], val)
            plsc.addupdate_scatter(ht, [ytv + lane_off], ones)

    def _reduce_chunk(chunk, _):
        o = chunk * _L
        vp = hc[pl.ds(o, _L)]
        cp0 = vp & 0xFFFF
        ci0 = lax.shift_right_logical(vp, 16)
        ct0 = ht[pl.ds(o, _L)]

        def _acc(r, carry):
            cp, ct, ci = carry
            v = hc[pl.ds(r * _STRIDE + o, _L)]
            cp = cp + (v & 0xFFFF)
            ci = ci + lax.shift_right_logical(v, 16)
            ct = ct + ht[pl.ds(r * _STRIDE + o, _L)]
            return cp, ct, ci
        cp, ct, ci = lax.fori_loop(1, _L, _acc, (cp0, ct0, ci0))
        red[pl.ds(o, _L)] = cp
        red[pl.ds(_BINS + o, _L)] = ct
        red[pl.ds(2 * _BINS + o, _L)] = ci
        return 0
    lax.fori_loop(0, _BINS // _L, _reduce_chunk, 0)

    pltpu.sync_copy(red, shared.at[pl.ds(s * _RED, _RED)])
    plsc.subcore_barrier()

    @pl.when(s == 0)
    def _combine():
        pltpu.sync_copy(shared, comb)
        fzeros = jnp.zeros((_L,), jnp.float32)

        def _sample(j, total):
            def _chunk(chunk, carry):
                terms_acc, ct_acc = carry
                o = chunk * _L

                def _quarter(r, cnts):
                    cp, ct, ci = cnts
                    row = (4 * j + r) * _RED + o
                    cp = cp + comb[pl.ds(row, _L)]
                    ct = ct + comb[pl.ds(row + _BINS, _L)]
                    ci = ci + comb[pl.ds(row + 2 * _BINS, _L)]
                    return cp, ct, ci
                cp, ct, ci = lax.fori_loop(
                    0, 4, _quarter, (zeros, zeros, zeros))
                cpf = cp.astype(jnp.float32)
                ctf = ct.astype(jnp.float32)
                cif = ci.astype(jnp.float32)
                denom = cpf + ctf
                gidx = lane + o
                fg = gidx >= 1
                nz = (denom > 0.0) & fg
                safe = jnp.where(nz, denom, 1.0)
                terms_acc = terms_acc + jnp.where(
                    nz, 2.0 * ctf * cif / safe, 0.0)
                ct_acc = ct_acc + jnp.where(fg, ctf, 0.0)
                return terms_acc, ct_acc
            terms_acc, ct_acc = lax.fori_loop(
                0, _BINS // _L, _chunk, (fzeros, fzeros))
            inner = jnp.full((_L,), jnp.sum(terms_acc), jnp.float32)
            ctsum = jnp.full((_L,), jnp.sum(ct_acc), jnp.float32)
            return total + inner / (ctsum * float(_N))
        total = lax.fori_loop(0, _NS // 4, _sample, fzeros)
        outv[...] = total
        pltpu.sync_copy(outv, out_hbm.at[pl.ds(c * _L, _L)])


@jax.jit
def kernel(y_pred, y_true):
    part = _dice_hist_sc(y_pred, y_true)
    return (1.0 - part[0] - part[_L]).astype(jnp.float32)


# speedup vs baseline: 1.0226x; 1.0226x over previous
import functools

import jax
import jax.numpy as jnp
from jax import lax
from jax.experimental import pallas as pl
from jax.experimental.pallas import tpu as pltpu
from jax.experimental.pallas import tpu_sc as plsc

_NUM_CLASSES = 150
_N = 8
_S = 512
_HW = _S * _S
_TOTAL = _N * _HW
_NC, _NS = 2, 16
_E = _TOTAL // (_NC * _NS)
_ROWS = 32
_CH = _ROWS * _S
_NCHUNK = _E // _CH
_L = 16
_BINS = 160
_STRIDE = 161
_HSZ = _L * _STRIDE
_RED = 3 * _BINS

_mesh = plsc.VectorSubcoreMesh(core_axis_name="c", subcore_axis_name="s")


@functools.partial(
    pl.kernel,
    out_type=jax.ShapeDtypeStruct((_NC * _L,), jnp.float32),
    mesh=_mesh,
    compiler_params=pltpu.CompilerParams(
        needs_layout_passes=False, use_tc_tiling_on_sc=True),
    scratch_types=[
        pltpu.VMEM((_ROWS, _S), jnp.int32),
        pltpu.VMEM((_ROWS, _S), jnp.int32),
        pltpu.VMEM((_ROWS, _S), jnp.int32),
        pltpu.VMEM((_ROWS, _S), jnp.int32),
        pltpu.VMEM((_HSZ,), jnp.int32),
        pltpu.VMEM((_HSZ,), jnp.int32),
        pltpu.VMEM((_RED,), jnp.int32),
        pltpu.VMEM_SHARED((_NS * _RED,), jnp.int32),
        pltpu.VMEM((4 * _RED,), jnp.int32),
        pltpu.VMEM_SHARED((4 * _L,), jnp.float32),
        pltpu.VMEM((4 * _L,), jnp.float32),
        pltpu.VMEM((_L,), jnp.float32),
        pltpu.SemaphoreType.DMA,
        pltpu.SemaphoreType.DMA,
        pltpu.SemaphoreType.DMA,
        pltpu.SemaphoreType.DMA,
    ],
)
def _dice_hist_sc(yp_hbm, yt_hbm, out_hbm, ypb0, ypb1, ytb0, ytb1, hc, ht,
                  red, shared, comb, terms, termsl, outv, sp0, sp1, st0, st1):
    c = lax.axis_index("c")
    s = lax.axis_index("s")
    g = c * _NS + s
    n = g >> 2
    row0 = (g & 3) * (_S // 4)

    lane = lax.iota(jnp.int32, _L)
    lane_off = lane * _STRIDE
    ones = jnp.ones((_L,), jnp.int32)
    zeros = jnp.zeros((_L,), jnp.int32)

    bufs = ((ypb0, ytb0, sp0, st0), (ypb1, ytb1, sp1, st1))

    def _issue(k):
        ypb, ytb, sp, st = bufs[k % 2]
        r = row0 + k * _ROWS
        return (
            pltpu.async_copy(yp_hbm.at[n, 0, pl.ds(r, _ROWS), :], ypb, sp),
            pltpu.async_copy(yt_hbm.at[n, 0, pl.ds(r, _ROWS), :], ytb, st))

    pending = _issue(0)

    @plsc.parallel_loop(0, _HSZ // _L, unroll=8)
    def _zero(i):
        hc[pl.ds(i * _L, _L)] = zeros
        ht[pl.ds(i * _L, _L)] = zeros

    for k in range(_NCHUNK):
        ypb, ytb, _, _ = bufs[k % 2]
        hp_, ht_ = pending
        hp_.wait()
        ht_.wait()
        if k + 1 < _NCHUNK:
            pending = _issue(k + 1)

        @plsc.parallel_loop(0, _CH // _L, unroll=8)
        def _body(i):
            r = i >> 5
            col = (i & 31) * _L
            ypv = ypb[r, pl.ds(col, _L)]
            ytv = ytb[r, pl.ds(col, _L)]
            val = jnp.where(ypv == ytv, jnp.int32(0x10001), jnp.int32(1))
            plsc.addupdate_scatter(hc, [ypv + lane_off], val)
            plsc.addupdate_scatter(ht, [ytv + lane_off], ones)

    def _reduce_chunk(chunk, _):
        o = chunk * _L
        vp = hc[pl.ds(o, _L)]
        cp0 = vp & 0xFFFF
        ci0 = lax.shift_right_logical(vp, 16)
        ct0 = ht[pl.ds(o, _L)]

        def _acc(r, carry):
            cp, ct, ci = carry
            v = hc[pl.ds(r * _STRIDE + o, _L)]
            cp = cp + (v & 0xFFFF)
            ci = ci + lax.shift_right_logical(v, 16)
            ct = ct + ht[pl.ds(r * _STRIDE + o, _L)]
            return cp, ct, ci
        cp, ct, ci = lax.fori_loop(1, _L, _acc, (cp0, ct0, ci0))
        red[pl.ds(o, _L)] = cp
        red[pl.ds(_BINS + o, _L)] = ct
        red[pl.ds(2 * _BINS + o, _L)] = ci
        return 0
    lax.fori_loop(0, _BINS // _L, _reduce_chunk, 0)

    pltpu.sync_copy(red, shared.at[pl.ds(s * _RED, _RED)])
    plsc.subcore_barrier()

    @pl.when(s < 4)
    def _combine():
        pltpu.sync_copy(shared.at[pl.ds(4 * s * _RED, 4 * _RED)], comb)
        fzeros = jnp.zeros((_L,), jnp.float32)

        def _chunk(chunk, carry):
            terms_acc, ct_acc = carry
            o = chunk * _L

            def _quarter(r, cnts):
                cp, ct, ci = cnts
                row = r * _RED + o
                cp = cp + comb[pl.ds(row, _L)]
                ct = ct + comb[pl.ds(row + _BINS, _L)]
                ci = ci + comb[pl.ds(row + 2 * _BINS, _L)]
                return cp, ct, ci
            cp, ct, ci = lax.fori_loop(
                0, 4, _quarter, (zeros, zeros, zeros))
            cpf = cp.astype(jnp.float32)
            ctf = ct.astype(jnp.float32)
            cif = ci.astype(jnp.float32)
            denom = cpf + ctf
            gidx = lane + o
            fg = gidx >= 1
            nz = (denom > 0.0) & fg
            safe = jnp.where(nz, denom, 1.0)
            terms_acc = terms_acc + jnp.where(
                nz, 2.0 * ctf * cif / safe, 0.0)
            ct_acc = ct_acc + jnp.where(fg, ctf, 0.0)
            return terms_acc, ct_acc
        terms_acc, ct_acc = lax.fori_loop(
            0, _BINS // _L, _chunk, (fzeros, fzeros))
        inner = jnp.full((_L,), jnp.sum(terms_acc), jnp.float32)
        ctsum = jnp.full((_L,), jnp.sum(ct_acc), jnp.float32)
        outv[...] = inner / (ctsum * float(_N))
        pltpu.sync_copy(outv, terms.at[pl.ds(s * _L, _L)])

    plsc.subcore_barrier()

    @pl.when(s == 0)
    def _final():
        pltpu.sync_copy(terms, termsl)
        total = (termsl[pl.ds(0, _L)] + termsl[pl.ds(_L, _L)]
                 + termsl[pl.ds(2 * _L, _L)] + termsl[pl.ds(3 * _L, _L)])
        outv[...] = total
        pltpu.sync_copy(outv, out_hbm.at[pl.ds(c * _L, _L)])


@jax.jit
def kernel(y_pred, y_true):
    part = _dice_hist_sc(y_pred, y_true)
    return (1.0 - part[0] - part[_L]).astype(jnp.float32)
